# Initial kernel scaffold; baseline (speedup 1.0000x reference)
import jax
import jax.numpy as jnp
from jax.experimental import pallas as pl


def kernel(x, idx0, d_row0, d_col0, d_val0, u_row0, u_col0, u_val0, idx1, d_row1, d_col1, d_val1, u_row1, u_col1, u_val1, idx2, d_row2, d_col2, d_val2, u_row2, u_col2, u_val2, idx3, d_row3, d_col3, d_val3, u_row3, u_col3, u_val3, enc_W0, enc_b0, enc_W1, enc_b1, enc_W2, enc_b2, enc_W3, enc_b3, lat_W, lat_b, dec0_W, dec0_b, dec_W1, dec_b1, dec_W2, dec_b2, dec_W3, dec_b3, dec_W4, dec_b4, dec_W5, dec_b5):
    raise NotImplementedError("write your pallas kernel here")


# speedup vs baseline: 9.8536x; 9.8536x over previous
import functools

import jax
import jax.numpy as jnp
from jax import lax
from jax.experimental import pallas as pl
from jax.experimental.pallas import tpu as pltpu
from jax.experimental.pallas import tpu_sc as plsc

S = 9
N = [50000, 12500, 3125, 781, 195]
OC = [32, 32, 32, 64]
IN_CH = 3
LATENT = 128
B = 4

NC = 2
NS = 16
NW = NC * NS

BLK = 256


def _pad_rows(n):
    return ((n + 255) // 256) * 256


NP = [_pad_rows(n) for n in N]


@functools.lru_cache(maxsize=None)
def _sc_gather(rows, lanes, m_total):
    assert lanes % 128 == 0
    assert m_total % 128 == 0
    n_chunks = m_total // 128
    GRP = 8
    sub = max(1, min(GRP, 65536 // (128 * lanes)))
    n_g, tail = divmod(n_chunks, GRP)
    cnt_max = -(-n_g // NW)

    mesh = plsc.VectorSubcoreMesh(core_axis_name="c", subcore_axis_name="s")

    @functools.partial(
        pl.kernel,
        out_type=jax.ShapeDtypeStruct((m_total, lanes), jnp.float32),
        mesh=mesh,
        scratch_types=[
            pltpu.VMEM((GRP, 128), jnp.int32),
            pltpu.VMEM((sub * 128, lanes), jnp.float32),
            pltpu.SemaphoreType.DMA,
        ],
    )
    def k(table_hbm, idx_hbm, out_hbm, idx_v, rows_v, sem):
        wid = lax.axis_index("s") * NC + lax.axis_index("c")

        def do_chunks(idx_lo, n_here, out_row0):
            done = 0
            while done < n_here:
                bb = min(sub, n_here - done)
                cps = []
                for i in range(bb):
                    cps.append(
                        pltpu.async_copy(
                            table_hbm.at[idx_v.at[idx_lo + done + i]],
                            rows_v.at[pl.ds(i * 128, 128)],
                            sem,
                        )
                    )
                for c in cps:
                    c.wait()
                pltpu.sync_copy(
                    rows_v.at[pl.ds(0, bb * 128)],
                    out_hbm.at[pl.ds(out_row0 + done * 128, bb * 128)],
                )
                done += bb

        for kk in range(cnt_max):
            g = wid + kk * NW

            @pl.when(g < n_g)
            def _():
                pltpu.sync_copy(idx_hbm.at[pl.ds(g * GRP, GRP)], idx_v)
                do_chunks(0, GRP, g * GRP * 128)

        if tail:

            @pl.when(wid == 0)
            def _():
                pltpu.sync_copy(
                    idx_hbm.at[pl.ds(n_g * GRP, tail)],
                    idx_v.at[pl.ds(0, tail)],
                )
                do_chunks(0, tail, n_g * GRP * 128)

    return k


def _act(y, act):
    if act == "elu":
        return jnp.where(y > 0, y, jnp.exp(y) - 1.0)
    if act == "sigmoid":
        return 1.0 / (1.0 + jnp.exp(-y))
    return y


@functools.lru_cache(maxsize=None)
def _tc_matmul(rows, k_dim, oc_l, act):
    def body(g_ref, w_ref, b_ref, o_ref):
        y = jnp.dot(g_ref[...], w_ref[...], preferred_element_type=jnp.float32)
        o_ref[...] = _act(y + b_ref[...], act)

    return pl.pallas_call(
        body,
        grid=(rows // BLK,),
        in_specs=[
            pl.BlockSpec((BLK, k_dim), lambda i: (i, 0)),
            pl.BlockSpec((k_dim, oc_l), lambda i: (0, 0)),
            pl.BlockSpec((1, oc_l), lambda i: (0, 0)),
        ],
        out_specs=pl.BlockSpec((BLK, oc_l), lambda i: (i, 0)),
        out_shape=jax.ShapeDtypeStruct((rows, oc_l), jnp.float32),
    )


@functools.lru_cache(maxsize=None)
def _tc_pool_madd(rows, lanes):
    def body(g_ref, v_ref, o_ref):
        g = g_ref[...]
        v = v_ref[...]
        o_ref[...] = (
            v[0][:, None] * g[0] + v[1][:, None] * g[1] + v[2][:, None] * g[2]
        )

    return pl.pallas_call(
        body,
        grid=(rows // BLK,),
        in_specs=[
            pl.BlockSpec((3, BLK, lanes), lambda i: (0, i, 0)),
            pl.BlockSpec((3, BLK), lambda i: (0, i)),
        ],
        out_specs=pl.BlockSpec((BLK, lanes), lambda i: (i, 0)),
        out_shape=jax.ShapeDtypeStruct((rows, lanes), jnp.float32),
    )


@functools.lru_cache(maxsize=None)
def _tc_dense(m_rows, k_dim, n_dim, act):
    def body(x_ref, w_ref, b_ref, o_ref):
        y = jnp.dot(x_ref[...], w_ref[...], preferred_element_type=jnp.float32)
        o_ref[...] = _act(y + b_ref[...], act)

    return pl.pallas_call(
        body,
        out_shape=jax.ShapeDtypeStruct((m_rows, n_dim), jnp.float32),
    )


def _prep_conv_idx(idx, n_pad):
    n = idx.shape[0]
    ip = jnp.zeros((n_pad, S), jnp.int32).at[:n, :].set(idx)
    return ip.reshape(n_pad * S // 128, 128)


def _prep_pool(col, val, n_out, n_pad):
    c = col.reshape(n_out, 3).T
    v = val.reshape(n_out, 3).T
    cp = jnp.zeros((3, n_pad), jnp.int32).at[:, :n_out].set(c)
    vp = jnp.zeros((3, n_pad), jnp.float32).at[:, :n_out].set(v)
    return cp.reshape(3 * n_pad // 128, 128), vp


def _expand_w(w, cin, cin_p, oc, oc_p):
    w3 = w.reshape(S, cin, oc)
    w3 = jnp.zeros((S, cin_p, oc_p), jnp.float32).at[:, :cin, :oc].set(w3)
    eye = jnp.eye(B, dtype=jnp.float32)
    wx = eye[None, :, None, :, None] * w3[:, None, :, None, :]
    return wx.reshape(S * B * cin_p, B * oc_p)


def _expand_b(bias, oc, oc_p):
    bp = jnp.zeros((oc_p,), jnp.float32).at[:oc].set(bias)
    return jnp.tile(bp, B).reshape(1, B * oc_p)


def _conv(h, idx_flat, wx, bx, n_pad, act):
    lanes = h.shape[1]
    g = _sc_gather(h.shape[0], lanes, n_pad * S)(h, idx_flat)
    g = g.reshape(n_pad, S * lanes)
    return _tc_matmul(n_pad, S * lanes, wx.shape[1], act)(g, wx, bx)


def _pool(h, col_flat, val2, n_pad):
    lanes = h.shape[1]
    g = _sc_gather(h.shape[0], lanes, 3 * n_pad)(h, col_flat)
    g = g.reshape(3, n_pad, lanes)
    return _tc_pool_madd(n_pad, lanes)(g, val2)


def kernel(x, idx0, d_row0, d_col0, d_val0, u_row0, u_col0, u_val0,
           idx1, d_row1, d_col1, d_val1, u_row1, u_col1, u_val1,
           idx2, d_row2, d_col2, d_val2, u_row2, u_col2, u_val2,
           idx3, d_row3, d_col3, d_val3, u_row3, u_col3, u_val3,
           enc_W0, enc_b0, enc_W1, enc_b1, enc_W2, enc_b2, enc_W3, enc_b3,
           lat_W, lat_b, dec0_W, dec0_b,
           dec_W1, dec_b1, dec_W2, dec_b2, dec_W3, dec_b3, dec_W4, dec_b4,
           dec_W5, dec_b5):
    idxs = [idx0, idx1, idx2, idx3]
    d_cols = [d_col0, d_col1, d_col2, d_col3]
    d_vals = [d_val0, d_val1, d_val2, d_val3]
    u_cols = [u_col0, u_col1, u_col2, u_col3]
    u_vals = [u_val0, u_val1, u_val2, u_val3]
    enc_W = [enc_W0, enc_W1, enc_W2, enc_W3]
    enc_b = [enc_b0, enc_b1, enc_b2, enc_b3]
    dec_W = [dec_W1, dec_W2, dec_W3, dec_W4]
    dec_b = [dec_b1, dec_b2, dec_b3, dec_b4]

    idx_flat = [_prep_conv_idx(idxs[i], NP[i]) for i in range(4)]

    xt = x.transpose(1, 0, 2)
    h = jnp.zeros((N[0], B, 32), jnp.float32).at[:, :, :IN_CH].set(xt)
    h = h.reshape(N[0], B * 32)

    cin, cin_p = IN_CH, 32
    for i in range(4):
        wx = _expand_w(enc_W[i], cin, cin_p, OC[i], OC[i])
        bx = _expand_b(enc_b[i], OC[i], OC[i])
        h = _conv(h, idx_flat[i], wx, bx, NP[i], "elu")
        colf, val2 = _prep_pool(d_cols[i], d_vals[i], N[i + 1], NP[i + 1])
        h = _pool(h, colf, val2, NP[i + 1])
        cin = cin_p = OC[i]

    hf = h[: N[4]].reshape(N[4], B, OC[-1]).transpose(1, 0, 2)
    hf = hf.reshape(B, N[4] * OC[-1])
    hf = jnp.zeros((8, N[4] * OC[-1]), jnp.float32).at[:B].set(hf)
    mu = _tc_dense(8, N[4] * OC[-1], LATENT, "sigmoid")(
        hf, lat_W, lat_b.reshape(1, -1)
    )
    z = _tc_dense(8, LATENT, N[4] * OC[-1], "none")(
        mu, dec0_W, dec0_b.reshape(1, -1)
    )
    z = z[:B].reshape(B, N[4], OC[-1]).transpose(1, 0, 2)
    h = jnp.zeros((NP[4], B, OC[-1]), jnp.float32).at[: N[4]].set(z)
    h = h.reshape(NP[4], B * OC[-1])

    for j, lvl in enumerate([3, 2, 1, 0]):
        colf, val2 = _prep_pool(u_cols[lvl], u_vals[lvl], N[lvl], NP[lvl])
        h = _pool(h, colf, val2, NP[lvl])
        cin = dec_W[j].shape[0] // S
        oc = dec_W[j].shape[1]
        wx = _expand_w(dec_W[j], cin, cin, oc, oc)
        bx = _expand_b(dec_b[j], oc, oc)
        h = _conv(h, idx_flat[lvl], wx, bx, NP[lvl], "elu")

    wx5 = _expand_w(dec_W5, OC[0], OC[0], IN_CH, 8)
    bx5 = _expand_b(dec_b5, IN_CH, 8)
    out = _conv(h, idx_flat[0], wx5, bx5, NP[0], "none")
    out = out[: N[0]].reshape(N[0], B, 8).transpose(1, 0, 2)
    return out[:, :, :IN_CH]
